# baseline (speedup 1.0000x reference)
import jax
import jax.numpy as jnp
from jax.experimental import pallas as pl
from jax.experimental.pallas import tpu as pltpu

CROP = 160
STRIDE = 4
THR = 0.2
K = 20
PAD = 96
PROWS = 720
PCOLS = 800
MAXOFF = 543


def _shift2(a, sy, sx, fill=0.0):
    n0, n1 = a.shape
    if sy > 0:
        a = jnp.concatenate([a[sy:], jnp.full((sy, n1), fill, a.dtype)], 0)
    elif sy < 0:
        a = jnp.concatenate([jnp.full((-sy, n1), fill, a.dtype), a[:sy]], 0)
    if sx > 0:
        a = jnp.concatenate([a[:, sx:], jnp.full((n0, sx), fill, a.dtype)], 1)
    elif sx < 0:
        a = jnp.concatenate([jnp.full((n0, -sx), fill, a.dtype), a[:, :sx]], 1)
    return a


def _im2col(x, kh, kw, stride):
    B, H, W, C = x.shape
    xp = jnp.pad(x, ((0, 0), (1, 3), (1, 3), (0, 0)))
    cols = []
    for u in range(kh):
        for v in range(kw):
            cols.append(xp[:, u:u + H:stride, v:v + W:stride, :])
    P = jnp.stack(cols, axis=3)
    return P.reshape(B, H // stride, W // stride, kh * kw * C)


def _dot1_body(p_ref, w_ref, b_ref, o_ref):
    z = jax.lax.dot_general(p_ref[...], w_ref[...], (((1,), (0,)), ((), ())),
                            precision=jax.lax.Precision.DEFAULT,
                            preferred_element_type=jnp.float32)
    o_ref[...] = jnp.maximum(z + b_ref[...], 0.0)


def _dot2_body(p_ref, w_ref, o_ref):
    o_ref[...] = jax.lax.dot_general(
        p_ref[...], w_ref[...], (((1,), (0,)), ((), ())),
        precision=jax.lax.Precision.DEFAULT,
        preferred_element_type=jnp.float32)


def _peaks_body(z_ref, b2_ref, vals_ref, offs_ref, ip_ref, fp_ref):
    z = z_ref[0] + b2_ref[0, 0]
    cms = 1.0 / (1.0 + jnp.exp(-z))

    mp = cms
    for sy in (-1, 0, 1):
        for sx in (-1, 0, 1):
            if sy == 0 and sx == 0:
                continue
            mp = jnp.maximum(mp, _shift2(cms, sy, sx))
    keys = jnp.where((cms >= mp) & (cms > THR), cms, -1.0)

    rs = [_shift2(cms, 0, s) for s in (-2, -1, 0, 1, 2)]
    rowsum = rs[0] + rs[1] + rs[2] + rs[3] + rs[4]
    rowg = -2.0 * rs[0] - rs[1] + rs[3] + 2.0 * rs[4]
    cs = [_shift2(rowsum, s, 0) for s in (-2, -1, 0, 1, 2)]
    SUM = cs[0] + cs[1] + cs[2] + cs[3] + cs[4]
    GY = -2.0 * cs[0] - cs[1] + cs[3] + 2.0 * cs[4]
    gs = [_shift2(rowg, s, 0) for s in (-2, -1, 0, 1, 2)]
    GX = gs[0] + gs[1] + gs[2] + gs[3] + gs[4]

    lin = (jax.lax.broadcasted_iota(jnp.int32, (128, 128), 0) * 128
           + jax.lax.broadcasted_iota(jnp.int32, (128, 128), 1))

    for t in range(K):
        m = jnp.max(keys)
        sel = jnp.min(jnp.where(keys == m, lin, jnp.int32(1 << 30)))
        oh = lin == sel
        sumv = jnp.sum(jnp.where(oh, SUM, 0.0))
        gxv = jnp.sum(jnp.where(oh, GX, 0.0))
        gyv = jnp.sum(jnp.where(oh, GY, 0.0))
        keys = jnp.where(oh, -2.0, keys)
        pyv = (sel // 128).astype(jnp.float32)
        pxv = (sel % 128).astype(jnp.float32)
        denom = sumv + 1e-8
        ptx = (pxv + gxv / denom) * float(STRIDE)
        pty = (pyv + gyv / denom) * float(STRIDE)
        vals_ref[0, 0, t] = jnp.where(m > 0.0, m, 0.0)
        offs_ref[0, t, 0] = ptx - CROP / 2.0
        offs_ref[0, t, 1] = pty - CROP / 2.0
        x1 = ptx - (CROP - 1) / 2.0
        y1 = pty - (CROP - 1) / 2.0
        x0f = jnp.floor(x1)
        y0f = jnp.floor(y1)
        ip_ref[0, t, 0] = jnp.clip(y0f.astype(jnp.int32) + PAD, 0, MAXOFF)
        ip_ref[0, t, 1] = jnp.clip(x0f.astype(jnp.int32) + PAD, 0, MAXOFF)
        fp_ref[0, t, 0] = y1 - y0f
        fp_ref[0, t, 1] = x1 - x0f


def _crop_body(ip_ref, fp_ref, p00_ref, p01_ref, p10_ref, p11_ref, out_ref):
    b = pl.program_id(0)
    t = pl.program_id(1)
    k = b * K + t
    iy = ip_ref[k, 0]
    ix = ip_ref[k, 1]
    wy = fp_ref[k, 0]
    wx = fp_ref[k, 1]
    iy8 = pl.multiple_of((iy // 8) * 8, 8)
    ix128 = pl.multiple_of((ix // 128) * 128, 128)
    shy = (168 - (iy - iy8)) % 168
    shx = (288 - (ix - ix128)) % 288

    def win(ref):
        w = ref[0, pl.ds(iy8, 168), pl.ds(ix128, 288)]
        w = pltpu.roll(w, shy, 0)
        w = pltpu.roll(w, shx, 1)
        return w[:CROP, :CROP]

    a = win(p00_ref)
    bw = win(p01_ref)
    c = win(p10_ref)
    d = win(p11_ref)
    out_ref[0, 0] = ((1.0 - wy) * ((1.0 - wx) * a + wx * bw)
                     + wy * ((1.0 - wx) * c + wx * d))


def _run_dot1(P1, w1f, b1r):
    M = P1.shape[0]
    CH = 8192
    return pl.pallas_call(
        _dot1_body,
        grid=(M // CH,),
        in_specs=[pl.BlockSpec((CH, 25), lambda i: (i, 0)),
                  pl.BlockSpec((25, 16), lambda i: (0, 0)),
                  pl.BlockSpec((1, 16), lambda i: (0, 0))],
        out_specs=pl.BlockSpec((CH, 16), lambda i: (i, 0)),
        out_shape=jax.ShapeDtypeStruct((M, 16), jnp.float32),
    )(P1, w1f, b1r)


def _run_dot2(P2, w2f):
    M = P2.shape[0]
    CH = 4096
    return pl.pallas_call(
        _dot2_body,
        grid=(M // CH,),
        in_specs=[pl.BlockSpec((CH, 400), lambda i: (i, 0)),
                  pl.BlockSpec((400, 128), lambda i: (0, 0))],
        out_specs=pl.BlockSpec((CH, 128), lambda i: (i, 0)),
        out_shape=jax.ShapeDtypeStruct((M, 128), jnp.float32),
    )(P2, w2f)


def _run_peaks(z2, b2r):
    B = z2.shape[0]
    f32 = jnp.float32
    return pl.pallas_call(
        _peaks_body,
        grid=(B,),
        in_specs=[
            pl.BlockSpec((1, 128, 128), lambda b: (b, 0, 0)),
            pl.BlockSpec(memory_space=pltpu.SMEM),
        ],
        out_specs=[
            pl.BlockSpec((1, 1, K), lambda b: (b, 0, 0), memory_space=pltpu.SMEM),
            pl.BlockSpec((1, K, 2), lambda b: (b, 0, 0), memory_space=pltpu.SMEM),
            pl.BlockSpec((1, K, 2), lambda b: (b, 0, 0), memory_space=pltpu.SMEM),
            pl.BlockSpec((1, K, 2), lambda b: (b, 0, 0), memory_space=pltpu.SMEM),
        ],
        out_shape=[
            jax.ShapeDtypeStruct((B, 1, K), f32),
            jax.ShapeDtypeStruct((B, K, 2), f32),
            jax.ShapeDtypeStruct((B, K, 2), jnp.int32),
            jax.ShapeDtypeStruct((B, K, 2), f32),
        ],
    )(z2, b2r)


def _run_crop(ip, fp, p00, p01, p10, p11):
    B = p00.shape[0]
    img_spec = pl.BlockSpec((1, PROWS, PCOLS), lambda b, t: (b, 0, 0))
    return pl.pallas_call(
        _crop_body,
        grid=(B, K),
        in_specs=[
            pl.BlockSpec(memory_space=pltpu.SMEM),
            pl.BlockSpec(memory_space=pltpu.SMEM),
            img_spec, img_spec, img_spec, img_spec,
        ],
        out_specs=pl.BlockSpec((1, 1, CROP, CROP), lambda b, t: (b, t, 0, 0)),
        out_shape=jax.ShapeDtypeStruct((B, K, CROP, CROP), jnp.float32),
    )(ip, fp, p00, p01, p10, p11)


def kernel(full_imgs, W1, b1, W2, b2):
    imgs = full_imgs[..., 0]
    B = imgs.shape[0]

    P1 = _im2col(full_imgs, 5, 5, 2).reshape(B * 256 * 256, 25)
    h = _run_dot1(P1, W1.reshape(25, 16), b1.reshape(1, 16))
    h4 = h.reshape(B, 256, 256, 16)

    P2 = _im2col(h4, 5, 5, 2).reshape(B * 128 * 128, 400)
    w2f = jnp.pad(W2.reshape(400, 1), ((0, 0), (0, 127)))
    z2 = _run_dot2(P2, w2f)[:, 0].reshape(B, 128, 128)

    vals, offs, ip, fp = _run_peaks(z2, b2.reshape(1, 1))

    def _padv(x, row1, col1):
        r = x[:, 1:, :] if row1 else x
        r = jnp.pad(r, ((0, 0), (PAD, PROWS - PAD - r.shape[1]), (0, 0)),
                    mode='edge')
        c = r[:, :, 1:] if col1 else r
        return jnp.pad(c, ((0, 0), (0, 0), (PAD, PCOLS - PAD - c.shape[2])),
                       mode='edge')

    crops = _run_crop(ip.reshape(B * K, 2), fp.reshape(B * K, 2),
                      _padv(imgs, False, False), _padv(imgs, False, True),
                      _padv(imgs, True, False), _padv(imgs, True, True))

    return (crops.reshape(B * K, CROP, CROP, 1),
            offs.reshape(B * K, 2),
            vals.reshape(B * K))


# speedup vs baseline: 4.2597x; 4.2597x over previous
import jax
import jax.numpy as jnp
from jax.experimental import pallas as pl
from jax.experimental.pallas import tpu as pltpu

CROP = 160
STRIDE = 4
THR = 0.2
K = 20
PAD = 96
PROWS = 720
PCOLS = 800
MAXOFF = 543


def _shift2(a, sy, sx, fill=0.0):
    n0, n1 = a.shape[-2:]
    lead = a.shape[:-2]
    if sy > 0:
        a = jnp.concatenate([a[..., sy:, :],
                             jnp.full((*lead, sy, n1), fill, a.dtype)], -2)
    elif sy < 0:
        a = jnp.concatenate([jnp.full((*lead, -sy, n1), fill, a.dtype),
                             a[..., :sy, :]], -2)
    if sx > 0:
        a = jnp.concatenate([a[..., :, sx:],
                             jnp.full((*lead, n0, sx), fill, a.dtype)], -1)
    elif sx < 0:
        a = jnp.concatenate([jnp.full((*lead, n0, -sx), fill, a.dtype),
                             a[..., :, :sx]], -1)
    return a


def _taps():
    for u in range(5):
        e = u - 1
        for v in range(5):
            f = v - 1
            yield e % 2, e // 2, f % 2, f // 2


def _build_p1t(imgs):
    B = imgs.shape[0]
    xp = imgs.reshape(B, 256, 2, 256, 2).transpose(0, 2, 4, 1, 3)
    rows = [_shift2(xp[:, py, px], sy, sx).reshape(-1)
            for py, sy, px, sx in _taps()]
    return jnp.stack(rows, 0)


def _build_p2t(hT):
    B = hT.shape[1]
    hp = hT.reshape(16, B, 128, 2, 128, 2).transpose(3, 5, 0, 1, 2, 4)
    rows = [_shift2(hp[py, px], sy, sx).reshape(16, -1)
            for py, sy, px, sx in _taps()]
    return jnp.concatenate(rows, 0)


def _dott1_body(w_ref, p_ref, b_ref, o_ref):
    z = jax.lax.dot_general(w_ref[...], p_ref[...], (((1,), (0,)), ((), ())),
                            precision=jax.lax.Precision.DEFAULT,
                            preferred_element_type=jnp.float32)
    o_ref[...] = jnp.maximum(z + b_ref[...], 0.0)


def _dott2_body(w_ref, p_ref, o_ref):
    o_ref[...] = jax.lax.dot_general(
        w_ref[...], p_ref[...], (((1,), (0,)), ((), ())),
        precision=jax.lax.Precision.DEFAULT,
        preferred_element_type=jnp.float32)


def _run_dott1(w1T, P1T, b1c):
    M = P1T.shape[1]
    CH = 32768
    return pl.pallas_call(
        _dott1_body,
        grid=(M // CH,),
        in_specs=[pl.BlockSpec((16, 25), lambda i: (0, 0)),
                  pl.BlockSpec((25, CH), lambda i: (0, i)),
                  pl.BlockSpec((16, 1), lambda i: (0, 0))],
        out_specs=pl.BlockSpec((16, CH), lambda i: (0, i)),
        out_shape=jax.ShapeDtypeStruct((16, M), jnp.float32),
    )(w1T, P1T, b1c)


def _run_dott2(w2T, P2T):
    M = P2T.shape[1]
    CH = 8192
    return pl.pallas_call(
        _dott2_body,
        grid=(M // CH,),
        in_specs=[pl.BlockSpec((8, 400), lambda i: (0, 0)),
                  pl.BlockSpec((400, CH), lambda i: (0, i))],
        out_specs=pl.BlockSpec((8, CH), lambda i: (0, i)),
        out_shape=jax.ShapeDtypeStruct((8, M), jnp.float32),
    )(w2T, P2T)


def _peaks_body(z_ref, b2_ref, vals_ref, offs_ref, ip_ref, fp_ref):
    z = z_ref[0] + b2_ref[0, 0]
    cms = 1.0 / (1.0 + jnp.exp(-z))

    mp = cms
    for sy in (-1, 0, 1):
        for sx in (-1, 0, 1):
            if sy == 0 and sx == 0:
                continue
            mp = jnp.maximum(mp, _shift2(cms, sy, sx))
    keys = jnp.where((cms >= mp) & (cms > THR), cms, -1.0)

    rs = [_shift2(cms, 0, s) for s in (-2, -1, 0, 1, 2)]
    rowsum = rs[0] + rs[1] + rs[2] + rs[3] + rs[4]
    rowg = -2.0 * rs[0] - rs[1] + rs[3] + 2.0 * rs[4]
    cs = [_shift2(rowsum, s, 0) for s in (-2, -1, 0, 1, 2)]
    SUM = cs[0] + cs[1] + cs[2] + cs[3] + cs[4]
    GY = -2.0 * cs[0] - cs[1] + cs[3] + 2.0 * cs[4]
    gs = [_shift2(rowg, s, 0) for s in (-2, -1, 0, 1, 2)]
    GX = gs[0] + gs[1] + gs[2] + gs[3] + gs[4]

    lin = (jax.lax.broadcasted_iota(jnp.int32, (128, 128), 0) * 128
           + jax.lax.broadcasted_iota(jnp.int32, (128, 128), 1))

    for t in range(K):
        m = jnp.max(keys)
        sel = jnp.min(jnp.where(keys == m, lin, jnp.int32(1 << 30)))
        oh = lin == sel
        sumv = jnp.sum(jnp.where(oh, SUM, 0.0))
        gxv = jnp.sum(jnp.where(oh, GX, 0.0))
        gyv = jnp.sum(jnp.where(oh, GY, 0.0))
        keys = jnp.where(oh, -2.0, keys)
        pyv = (sel // 128).astype(jnp.float32)
        pxv = (sel % 128).astype(jnp.float32)
        denom = sumv + 1e-8
        ptx = (pxv + gxv / denom) * float(STRIDE)
        pty = (pyv + gyv / denom) * float(STRIDE)
        vals_ref[0, 0, t] = jnp.where(m > 0.0, m, 0.0)
        offs_ref[0, t, 0] = ptx - CROP / 2.0
        offs_ref[0, t, 1] = pty - CROP / 2.0
        x1 = ptx - (CROP - 1) / 2.0
        y1 = pty - (CROP - 1) / 2.0
        x0f = jnp.floor(x1)
        y0f = jnp.floor(y1)
        ip_ref[0, t, 0] = jnp.clip(y0f.astype(jnp.int32) + PAD, 0, MAXOFF)
        ip_ref[0, t, 1] = jnp.clip(x0f.astype(jnp.int32) + PAD, 0, MAXOFF)
        fp_ref[0, t, 0] = y1 - y0f
        fp_ref[0, t, 1] = x1 - x0f


def _run_peaks(z2, b2r):
    B = z2.shape[0]
    f32 = jnp.float32
    return pl.pallas_call(
        _peaks_body,
        grid=(B,),
        in_specs=[
            pl.BlockSpec((1, 128, 128), lambda b: (b, 0, 0)),
            pl.BlockSpec(memory_space=pltpu.SMEM),
        ],
        out_specs=[
            pl.BlockSpec((1, 1, K), lambda b: (b, 0, 0), memory_space=pltpu.SMEM),
            pl.BlockSpec((1, K, 2), lambda b: (b, 0, 0), memory_space=pltpu.SMEM),
            pl.BlockSpec((1, K, 2), lambda b: (b, 0, 0), memory_space=pltpu.SMEM),
            pl.BlockSpec((1, K, 2), lambda b: (b, 0, 0), memory_space=pltpu.SMEM),
        ],
        out_shape=[
            jax.ShapeDtypeStruct((B, 1, K), f32),
            jax.ShapeDtypeStruct((B, K, 2), f32),
            jax.ShapeDtypeStruct((B, K, 2), jnp.int32),
            jax.ShapeDtypeStruct((B, K, 2), f32),
        ],
    )(z2, b2r)


def _crop_body(ip_ref, fp_ref, p00_ref, p01_ref, p10_ref, p11_ref, out_ref):
    b = pl.program_id(0)
    t = pl.program_id(1)
    k = b * K + t
    iy = ip_ref[k, 0]
    ix = ip_ref[k, 1]
    wy = fp_ref[k, 0]
    wx = fp_ref[k, 1]
    iy8 = pl.multiple_of((iy // 8) * 8, 8)
    ix128 = pl.multiple_of((ix // 128) * 128, 128)
    shy = (168 - (iy - iy8)) % 168
    shx = (288 - (ix - ix128)) % 288

    def win(ref):
        w = ref[0, pl.ds(iy8, 168), pl.ds(ix128, 288)]
        w = pltpu.roll(w, shy, 0)
        w = pltpu.roll(w, shx, 1)
        return w[:CROP, :CROP]

    a = win(p00_ref)
    bw = win(p01_ref)
    c = win(p10_ref)
    d = win(p11_ref)
    out_ref[0, 0] = ((1.0 - wy) * ((1.0 - wx) * a + wx * bw)
                     + wy * ((1.0 - wx) * c + wx * d))


def _run_crop(ip, fp, p00, p01, p10, p11):
    B = p00.shape[0]
    img_spec = pl.BlockSpec((1, PROWS, PCOLS), lambda b, t: (b, 0, 0))
    return pl.pallas_call(
        _crop_body,
        grid=(B, K),
        in_specs=[
            pl.BlockSpec(memory_space=pltpu.SMEM),
            pl.BlockSpec(memory_space=pltpu.SMEM),
            img_spec, img_spec, img_spec, img_spec,
        ],
        out_specs=pl.BlockSpec((1, 1, CROP, CROP), lambda b, t: (b, t, 0, 0)),
        out_shape=jax.ShapeDtypeStruct((B, K, CROP, CROP), jnp.float32),
    )(ip, fp, p00, p01, p10, p11)


def kernel(full_imgs, W1, b1, W2, b2):
    imgs = full_imgs[..., 0]
    B = imgs.shape[0]

    P1T = _build_p1t(imgs)
    hT = _run_dott1(W1.reshape(25, 16).T, P1T, b1.reshape(16, 1))
    P2T = _build_p2t(hT.reshape(16, B, 256, 256))
    w2T = jnp.pad(W2.reshape(400, 1).T, ((0, 7), (0, 0)))
    z2 = _run_dott2(w2T, P2T)[0].reshape(B, 128, 128)

    vals, offs, ip, fp = _run_peaks(z2, b2.reshape(1, 1))

    def _padv(x, row1, col1):
        r = x[:, 1:, :] if row1 else x
        r = jnp.pad(r, ((0, 0), (PAD, PROWS - PAD - r.shape[1]), (0, 0)),
                    mode='edge')
        c = r[:, :, 1:] if col1 else r
        return jnp.pad(c, ((0, 0), (0, 0), (PAD, PCOLS - PAD - c.shape[2])),
                       mode='edge')

    crops = _run_crop(ip.reshape(B * K, 2), fp.reshape(B * K, 2),
                      _padv(imgs, False, False), _padv(imgs, False, True),
                      _padv(imgs, True, False), _padv(imgs, True, True))

    return (crops.reshape(B * K, CROP, CROP, 1),
            offs.reshape(B * K, 2),
            vals.reshape(B * K))


# speedup vs baseline: 19.2810x; 4.5263x over previous
import jax
import jax.numpy as jnp
from jax.experimental import pallas as pl
from jax.experimental.pallas import tpu as pltpu

CROP = 160
STRIDE = 4
THR = 0.2
K = 20
PAD = 96
PROWS = 720
PCOLS = 800
MAXOFF = 543


def _shift2(a, sy, sx, fill=0.0):
    n0, n1 = a.shape[-2:]
    lead = a.shape[:-2]
    if sy > 0:
        a = jnp.concatenate([a[..., sy:, :],
                             jnp.full((*lead, sy, n1), fill, a.dtype)], -2)
    elif sy < 0:
        a = jnp.concatenate([jnp.full((*lead, -sy, n1), fill, a.dtype),
                             a[..., :sy, :]], -2)
    if sx > 0:
        a = jnp.concatenate([a[..., :, sx:],
                             jnp.full((*lead, n0, sx), fill, a.dtype)], -1)
    elif sx < 0:
        a = jnp.concatenate([jnp.full((*lead, n0, -sx), fill, a.dtype),
                             a[..., :, :sx]], -1)
    return a


def _taps():
    for u in range(5):
        e = u - 1
        for v in range(5):
            f = v - 1
            yield e % 2, e // 2, f % 2, f // 2


def _build_p1t(imgs):
    B = imgs.shape[0]
    xp = imgs.reshape(B, 256, 2, 256, 2).transpose(0, 2, 4, 1, 3)
    rows = [_shift2(xp[:, py, px], sy, sx).reshape(-1)
            for py, sy, px, sx in _taps()]
    return jnp.stack(rows, 0).astype(jnp.bfloat16)


def _build_p2t(hT):
    B = hT.shape[1]
    hp = hT.reshape(16, B, 128, 2, 128, 2).transpose(3, 5, 0, 1, 2, 4)
    rows = [_shift2(hp[py, px], sy, sx).reshape(16, -1)
            for py, sy, px, sx in _taps()]
    return jnp.concatenate(rows, 0).astype(jnp.bfloat16)


def _dott1_body(w_ref, p_ref, b_ref, o_ref):
    z = jax.lax.dot_general(w_ref[...], p_ref[...], (((1,), (0,)), ((), ())),
                            precision=jax.lax.Precision.DEFAULT,
                            preferred_element_type=jnp.float32)
    o_ref[...] = jnp.maximum(z + b_ref[...], 0.0)


def _dott2_body(w_ref, p_ref, o_ref):
    o_ref[...] = jax.lax.dot_general(
        w_ref[...], p_ref[...], (((1,), (0,)), ((), ())),
        precision=jax.lax.Precision.DEFAULT,
        preferred_element_type=jnp.float32)


def _run_dott1(w1T, P1T, b1c):
    M = P1T.shape[1]
    CH = 32768
    return pl.pallas_call(
        _dott1_body,
        grid=(M // CH,),
        in_specs=[pl.BlockSpec((16, 25), lambda i: (0, 0)),
                  pl.BlockSpec((25, CH), lambda i: (0, i)),
                  pl.BlockSpec((16, 1), lambda i: (0, 0))],
        out_specs=pl.BlockSpec((16, CH), lambda i: (0, i)),
        out_shape=jax.ShapeDtypeStruct((16, M), jnp.float32),
    )(w1T, P1T, b1c)


def _run_dott2(w2T, P2T):
    M = P2T.shape[1]
    CH = 8192
    return pl.pallas_call(
        _dott2_body,
        grid=(M // CH,),
        in_specs=[pl.BlockSpec((8, 400), lambda i: (0, 0)),
                  pl.BlockSpec((400, CH), lambda i: (0, i))],
        out_specs=pl.BlockSpec((8, CH), lambda i: (0, i)),
        out_shape=jax.ShapeDtypeStruct((8, M), jnp.float32),
    )(w2T, P2T)


def _peaks_body(z_ref, b2_ref, vals_ref, offs_ref, ip_ref, fp_ref):
    z = z_ref[0] + b2_ref[0, 0]
    cms = 1.0 / (1.0 + jnp.exp(-z))

    mp = cms
    for sy in (-1, 0, 1):
        for sx in (-1, 0, 1):
            if sy == 0 and sx == 0:
                continue
            mp = jnp.maximum(mp, _shift2(cms, sy, sx))
    keys = jnp.where((cms >= mp) & (cms > THR), cms, -1.0)

    rs = [_shift2(cms, 0, s) for s in (-2, -1, 0, 1, 2)]
    rowsum = rs[0] + rs[1] + rs[2] + rs[3] + rs[4]
    rowg = -2.0 * rs[0] - rs[1] + rs[3] + 2.0 * rs[4]
    cs = [_shift2(rowsum, s, 0) for s in (-2, -1, 0, 1, 2)]
    SUM = cs[0] + cs[1] + cs[2] + cs[3] + cs[4]
    GY = -2.0 * cs[0] - cs[1] + cs[3] + 2.0 * cs[4]
    gs = [_shift2(rowg, s, 0) for s in (-2, -1, 0, 1, 2)]
    GX = gs[0] + gs[1] + gs[2] + gs[3] + gs[4]

    lin = (jax.lax.broadcasted_iota(jnp.int32, (128, 128), 0) * 128
           + jax.lax.broadcasted_iota(jnp.int32, (128, 128), 1))

    for t in range(K):
        m = jnp.max(keys)
        sel = jnp.min(jnp.where(keys == m, lin, jnp.int32(1 << 30)))
        oh = lin == sel
        sumv = jnp.sum(jnp.where(oh, SUM, 0.0))
        gxv = jnp.sum(jnp.where(oh, GX, 0.0))
        gyv = jnp.sum(jnp.where(oh, GY, 0.0))
        keys = jnp.where(oh, -2.0, keys)
        pyv = (sel // 128).astype(jnp.float32)
        pxv = (sel % 128).astype(jnp.float32)
        denom = sumv + 1e-8
        ptx = (pxv + gxv / denom) * float(STRIDE)
        pty = (pyv + gyv / denom) * float(STRIDE)
        vals_ref[0, 0, t] = jnp.where(m > 0.0, m, 0.0)
        offs_ref[0, t, 0] = ptx - CROP / 2.0
        offs_ref[0, t, 1] = pty - CROP / 2.0
        x1 = ptx - (CROP - 1) / 2.0
        y1 = pty - (CROP - 1) / 2.0
        x0f = jnp.floor(x1)
        y0f = jnp.floor(y1)
        ip_ref[0, t, 0] = jnp.clip(y0f.astype(jnp.int32) + PAD, 0, MAXOFF)
        ip_ref[0, t, 1] = jnp.clip(x0f.astype(jnp.int32) + PAD, 0, MAXOFF)
        fp_ref[0, t, 0] = y1 - y0f
        fp_ref[0, t, 1] = x1 - x0f


def _run_peaks(z2, b2r):
    B = z2.shape[0]
    f32 = jnp.float32
    return pl.pallas_call(
        _peaks_body,
        grid=(B,),
        in_specs=[
            pl.BlockSpec((1, 128, 128), lambda b: (b, 0, 0)),
            pl.BlockSpec(memory_space=pltpu.SMEM),
        ],
        out_specs=[
            pl.BlockSpec((1, 1, K), lambda b: (b, 0, 0), memory_space=pltpu.SMEM),
            pl.BlockSpec((1, K, 2), lambda b: (b, 0, 0), memory_space=pltpu.SMEM),
            pl.BlockSpec((1, K, 2), lambda b: (b, 0, 0), memory_space=pltpu.SMEM),
            pl.BlockSpec((1, K, 2), lambda b: (b, 0, 0), memory_space=pltpu.SMEM),
        ],
        out_shape=[
            jax.ShapeDtypeStruct((B, 1, K), f32),
            jax.ShapeDtypeStruct((B, K, 2), f32),
            jax.ShapeDtypeStruct((B, K, 2), jnp.int32),
            jax.ShapeDtypeStruct((B, K, 2), f32),
        ],
    )(z2, b2r)


def _crop_body(ip_ref, fp_ref, p00_ref, p01_ref, p10_ref, p11_ref, out_ref):
    b = pl.program_id(0)
    t = pl.program_id(1)
    k = b * K + t
    iy = ip_ref[k, 0]
    ix = ip_ref[k, 1]
    wy = fp_ref[k, 0]
    wx = fp_ref[k, 1]
    iy8 = pl.multiple_of((iy // 8) * 8, 8)
    ix128 = pl.multiple_of((ix // 128) * 128, 128)
    shy = (168 - (iy - iy8)) % 168
    shx = (288 - (ix - ix128)) % 288

    def win(ref):
        w = ref[0, pl.ds(iy8, 168), pl.ds(ix128, 288)]
        w = pltpu.roll(w, shy, 0)
        w = pltpu.roll(w, shx, 1)
        return w[:CROP, :CROP]

    a = win(p00_ref)
    bw = win(p01_ref)
    c = win(p10_ref)
    d = win(p11_ref)
    out_ref[0, 0] = ((1.0 - wy) * ((1.0 - wx) * a + wx * bw)
                     + wy * ((1.0 - wx) * c + wx * d))


def _run_crop(ip, fp, p00, p01, p10, p11):
    B = p00.shape[0]
    img_spec = pl.BlockSpec((1, PROWS, PCOLS), lambda b, t: (b, 0, 0))
    return pl.pallas_call(
        _crop_body,
        grid=(B, K),
        in_specs=[
            pl.BlockSpec(memory_space=pltpu.SMEM),
            pl.BlockSpec(memory_space=pltpu.SMEM),
            img_spec, img_spec, img_spec, img_spec,
        ],
        out_specs=pl.BlockSpec((1, 1, CROP, CROP), lambda b, t: (b, t, 0, 0)),
        out_shape=jax.ShapeDtypeStruct((B, K, CROP, CROP), jnp.float32),
    )(ip, fp, p00, p01, p10, p11)


def kernel(full_imgs, W1, b1, W2, b2):
    imgs = full_imgs[..., 0]
    B = imgs.shape[0]

    P1T = _build_p1t(imgs)
    w1T = W1.reshape(25, 16).T.astype(jnp.bfloat16)
    hT = _run_dott1(w1T, P1T, b1.reshape(16, 1))
    P2T = _build_p2t(hT.reshape(16, B, 256, 256))
    w2T = jnp.pad(W2.reshape(400, 1).T, ((0, 7), (0, 0))).astype(jnp.bfloat16)
    z2 = _run_dott2(w2T, P2T)[0].reshape(B, 128, 128)

    vals, offs, ip, fp = _run_peaks(z2, b2.reshape(1, 1))

    def _padv(x, row1, col1):
        r = x[:, 1:, :] if row1 else x
        r = jnp.pad(r, ((0, 0), (PAD, PROWS - PAD - r.shape[1]), (0, 0)),
                    mode='edge')
        c = r[:, :, 1:] if col1 else r
        return jnp.pad(c, ((0, 0), (0, 0), (PAD, PCOLS - PAD - c.shape[2])),
                       mode='edge')

    crops = _run_crop(ip.reshape(B * K, 2), fp.reshape(B * K, 2),
                      _padv(imgs, False, False), _padv(imgs, False, True),
                      _padv(imgs, True, False), _padv(imgs, True, True))

    return (crops.reshape(B * K, CROP, CROP, 1),
            offs.reshape(B * K, 2),
            vals.reshape(B * K))
